# baseline (speedup 1.0000x reference)
import jax
import jax.numpy as jnp
from jax import lax
from jax.experimental import pallas as pl
from jax.experimental.pallas import tpu as pltpu
from jax.experimental.pallas import tpu_sc as plsc

NC = 2
NS = 16
NW = NC * NS

B, T, D = 1024, 200, 128
N = B * T
NPW = N // NW
C = 128
NCHUNK = NPW // C
NB = 7
ROUNDS = -(-NCHUNK // NB)


def _body(W_hbm, idx_hbm, out_hbm, idx_v, buf_v, *sems):
    gsem = sems[:NB]
    wsem = sems[NB:]
    wid = lax.axis_index("s") * NC + lax.axis_index("c")
    base = wid * NPW
    pltpu.sync_copy(idx_hbm.at[wid], idx_v)

    def outer(r, carry):
        for b in range(NB):
            g = r * NB + b

            @pl.when(jnp.logical_and(g >= NB, g < NCHUNK))
            def _wait_prev_write(b=b):
                pltpu.make_async_copy(
                    buf_v.at[b], out_hbm.at[pl.ds(0, C), :], wsem[b]
                ).wait()

            @pl.when(g < NCHUNK)
            def _start_gather(b=b, g=g):
                pltpu.make_async_copy(
                    W_hbm.at[idx_v.at[g]], buf_v.at[b], gsem[b]
                ).start()

        for b in range(NB):
            g = r * NB + b

            @pl.when(g < NCHUNK)
            def _drain_gather_start_write(b=b, g=g):
                pltpu.make_async_copy(
                    W_hbm.at[idx_v.at[g]], buf_v.at[b], gsem[b]
                ).wait()
                pltpu.make_async_copy(
                    buf_v.at[b], out_hbm.at[pl.ds(base + g * C, C), :], wsem[b]
                ).start()

        return carry

    lax.fori_loop(0, ROUNDS, outer, 0)
    for b in range(NB):
        pltpu.make_async_copy(
            buf_v.at[b], out_hbm.at[pl.ds(0, C), :], wsem[b]
        ).wait()


@jax.jit
def _gather(W, idx3):
    mesh = plsc.VectorSubcoreMesh(core_axis_name="c", subcore_axis_name="s")
    f = pl.kernel(
        _body,
        mesh=mesh,
        out_type=jax.ShapeDtypeStruct((N, D), jnp.float32),
        scratch_types=[
            pltpu.VMEM((NCHUNK, C), jnp.int32),
            pltpu.VMEM((NB, C, D), jnp.float32),
        ]
        + [pltpu.SemaphoreType.DMA] * (2 * NB),
    )
    return f(W, idx3)


def kernel(W, indices):
    idx3 = indices.reshape(NW, NCHUNK, C)
    out = _gather(W, idx3)
    return out.reshape(B, T, D)


# speedup vs baseline: 2.8898x; 2.8898x over previous
import jax
import jax.numpy as jnp
from jax import lax
from jax.experimental import pallas as pl
from jax.experimental.pallas import tpu as pltpu
from jax.experimental.pallas import tpu_sc as plsc

NC = 2
NS = 16
NW = NC * NS

B, T, D = 1024, 200, 128
N = B * T
NPW = N // NW
C = 128
NCHUNK = NPW // C
NB = 7
ROUNDS = -(-NCHUNK // NB)


def _body(W_hbm, idx_hbm, out_hbm, idx_v, buf_v, table_sp, *sems):
    gsem = sems[:NB]
    wsem = sems[NB:]
    sid = lax.axis_index("s")
    wid = sid * NC + lax.axis_index("c")
    base = wid * NPW

    @pl.when(sid == 0)
    def _stage_table():
        pltpu.sync_copy(W_hbm, table_sp)

    pltpu.sync_copy(idx_hbm.at[wid], idx_v)
    plsc.subcore_barrier()

    def outer(r, carry):
        for b in range(NB):
            g = r * NB + b

            @pl.when(jnp.logical_and(g >= NB, g < NCHUNK))
            def _wait_prev_write(b=b):
                pltpu.make_async_copy(
                    buf_v.at[b], out_hbm.at[pl.ds(0, C), :], wsem[b]
                ).wait()

            @pl.when(g < NCHUNK)
            def _start_gather(b=b, g=g):
                pltpu.make_async_copy(
                    table_sp.at[idx_v.at[g]], buf_v.at[b], gsem[b]
                ).start()

        for b in range(NB):
            g = r * NB + b

            @pl.when(g < NCHUNK)
            def _drain_gather_start_write(b=b, g=g):
                pltpu.make_async_copy(
                    table_sp.at[idx_v.at[g]], buf_v.at[b], gsem[b]
                ).wait()
                pltpu.make_async_copy(
                    buf_v.at[b], out_hbm.at[pl.ds(base + g * C, C), :], wsem[b]
                ).start()

        return carry

    lax.fori_loop(0, ROUNDS, outer, 0)
    for b in range(NB):
        pltpu.make_async_copy(
            buf_v.at[b], out_hbm.at[pl.ds(0, C), :], wsem[b]
        ).wait()


@jax.jit
def _gather(W, idx3):
    mesh = plsc.VectorSubcoreMesh(core_axis_name="c", subcore_axis_name="s")
    f = pl.kernel(
        _body,
        mesh=mesh,
        out_type=jax.ShapeDtypeStruct((N, D), jnp.float32),
        scratch_types=[
            pltpu.VMEM((NCHUNK, C), jnp.int32),
            pltpu.VMEM((NB, C, D), jnp.float32),
            pltpu.VMEM_SHARED((1002, D), jnp.float32),
        ]
        + [pltpu.SemaphoreType.DMA] * (2 * NB),
    )
    return f(W, idx3)


def kernel(W, indices):
    idx3 = indices.reshape(NW, NCHUNK, C)
    out = _gather(W, idx3)
    return out.reshape(B, T, D)
